# Initial kernel scaffold; baseline (speedup 1.0000x reference)
import jax
import jax.numpy as jnp
from jax.experimental import pallas as pl


def kernel(inputs, embeddings, position_embeddings):
    raise NotImplementedError("write your pallas kernel here")


# speedup vs baseline: 1.1545x; 1.1545x over previous
import functools

import jax
import jax.numpy as jnp
from jax import lax
from jax.experimental import pallas as pl
from jax.experimental.pallas import tpu as pltpu
from jax.experimental.pallas import tpu_sc as plsc

NUM_CORES = 2
NUM_SUBCORES = 16
NUM_WORKERS = NUM_CORES * NUM_SUBCORES
LANES = 16


@functools.lru_cache(maxsize=None)
def _build(batch, seq, vocab, d_model):
    s_per_w = seq // NUM_WORKERS
    n_flat = batch * seq
    vregs_per_row = d_model // LANES

    mesh = plsc.VectorSubcoreMesh(core_axis_name="c", subcore_axis_name="s")

    @functools.partial(
        pl.kernel,
        mesh=mesh,
        out_type=jax.ShapeDtypeStruct((n_flat, d_model), jnp.float32),
        scratch_types=[
            pltpu.VMEM((s_per_w,), jnp.int32),
            pltpu.VMEM((s_per_w, d_model), jnp.float32),
            pltpu.VMEM((s_per_w, d_model), jnp.float32),
            pltpu.SemaphoreType.DMA,
        ],
    )
    def k(idx_hbm, emb_hbm, pos_hbm, out_hbm, idx_v, pos_v, g_v, sem):
        wid = lax.axis_index("s") * NUM_CORES + lax.axis_index("c")
        s_base = wid * s_per_w
        pltpu.sync_copy(pos_hbm.at[pl.ds(s_base, s_per_w)], pos_v)
        for b in range(batch):
            row0 = b * seq + s_base
            pltpu.sync_copy(idx_hbm.at[pl.ds(row0, s_per_w)], idx_v)
            pltpu.async_copy(emb_hbm.at[idx_v], g_v, sem).wait()

            def radd(r, _):
                for c in range(vregs_per_row):
                    sl = pl.ds(c * LANES, LANES)
                    g_v[r, sl] = g_v[r, sl] + pos_v[r, sl]
                return 0

            lax.fori_loop(0, s_per_w, radd, 0)
            pltpu.sync_copy(g_v, out_hbm.at[pl.ds(row0, s_per_w)])

    return k


def kernel(inputs, embeddings, position_embeddings):
    batch, seq = inputs.shape
    vocab, d_model = embeddings.shape
    idx_flat = inputs.reshape(-1).astype(jnp.int32)
    k = _build(batch, seq, vocab, d_model)
    out = k(idx_flat, embeddings, position_embeddings)
    return out.reshape(batch, seq, d_model)
